# Initial kernel scaffold; baseline (speedup 1.0000x reference)
import jax
import jax.numpy as jnp
from jax.experimental import pallas as pl


def kernel(tokens, pos_emb, tok_emb, ln1_g, ln1_b, lnf_g, lnf_b, wq_t, wk_t, wv_t, wo_t, cqw, cqb, ckw, ckb, cvw, cvb, wout_t):
    raise NotImplementedError("write your pallas kernel here")


# speedup vs baseline: 1.8507x; 1.8507x over previous
import jax
import jax.numpy as jnp
from jax import lax
from jax.experimental import pallas as pl
from jax.experimental.pallas import tpu as pltpu

L = 32
DIM = 4
NUM_HEADS = 2
HEAD_DIM = DIM // NUM_HEADS
VOCAB = 16
KSIZE = 3
SCALE = HEAD_DIM ** (-0.5)
LN_EPS = 1e-5

G = 16
GROUPS = G // 2
TOKB = G * L
GRP = 2 * L
ATT_W = NUM_HEADS * GRP
NEG = -1e30

PR_CW = 0
PR_CB = 3
PR_OUTB = 4
PR_WO = 5
PR_WOUT = 9


def _block_kernel(act_ref, par_ref, logits_ref, attn_ref):
    qkv_all = act_ref[:, 0:3 * DIM]
    x_all = act_ref[:, 3 * DIM:4 * DIM]

    cw = par_ref[PR_CW:PR_CW + KSIZE, 0:3 * DIM]
    cb = par_ref[PR_CB:PR_CB + 1, 0:3 * DIM]
    out_b = par_ref[PR_OUTB:PR_OUTB + 1, 0:VOCAB]
    wo = par_ref[PR_WO:PR_WO + DIM, 0:DIM]
    wout = par_ref[PR_WOUT:PR_WOUT + DIM, 0:VOCAB]

    pos = lax.broadcasted_iota(jnp.int32, (TOKB, 1), 0) % L
    zm1 = jnp.where(pos == 0, 0.0, pltpu.roll(qkv_all, shift=1, axis=0))
    zp1 = jnp.where(pos == L - 1, 0.0,
                    pltpu.roll(qkv_all, shift=TOKB - 1, axis=0))
    qkv_all = zm1 * cw[0:1, :] + qkv_all * cw[1:2, :] + zp1 * cw[2:3, :] + cb

    r_head = lax.broadcasted_iota(jnp.int32, (ATT_W, 2 * DIM), 0) // GRP
    c_head = (lax.broadcasted_iota(jnp.int32, (ATT_W, 2 * DIM), 1)
              % DIM) // HEAD_DIM
    kv_keep = r_head == c_head
    row_b = lax.broadcasted_iota(jnp.int32, (GRP, ATT_W), 0) // L
    row_l = lax.broadcasted_iota(jnp.int32, (GRP, ATT_W), 0) % L
    col_b = (lax.broadcasted_iota(jnp.int32, (GRP, ATT_W), 1) // L) % 2
    col_l = lax.broadcasted_iota(jnp.int32, (GRP, ATT_W), 1) % L
    bad = (col_b != row_b) | (col_l > row_l)

    def ln_core(z):
        mu = jnp.mean(z, axis=-1, keepdims=True)
        var = jnp.mean((z - mu) ** 2, axis=-1, keepdims=True)
        return (z - mu) * lax.rsqrt(var + LN_EPS)

    for g in range(GROUPS):
        r0 = g * GRP
        qkv = qkv_all[r0:r0 + GRP, :]
        x = x_all[r0:r0 + GRP, :]
        q_all = qkv[:, 0:DIM]
        kv = qkv[:, DIM:3 * DIM]
        kv_rep = jnp.concatenate([kv, kv], axis=0)
        kv_sel = jnp.where(kv_keep, kv_rep, 0.0)
        k_sel = kv_sel[:, 0:DIM]
        v_sel = kv_sel[:, DIM:2 * DIM]

        s = lax.dot_general(q_all, k_sel, (((1,), (1,)), ((), ())),
                            preferred_element_type=jnp.float32)
        s = jnp.where(bad, NEG, s)

        p_parts = []
        for h in range(NUM_HEADS):
            sg = s[:, h * GRP:(h + 1) * GRP]
            e = jnp.exp(sg - jnp.max(sg, axis=-1, keepdims=True))
            inv = pl.reciprocal(jnp.sum(e, axis=-1, keepdims=True),
                                approx=False)
            p_parts.append(e * inv)
        p = jnp.concatenate(p_parts, axis=1)

        for b in range(2):
            for h in range(NUM_HEADS):
                attn_ref[2 * g + b, h, :, :] = (
                    p_parts[h][b * L:(b + 1) * L, b * L:(b + 1) * L])

        ctxt = jnp.dot(p, v_sel, preferred_element_type=jnp.float32)
        x2 = x + jnp.dot(ctxt, wo, preferred_element_type=jnp.float32)
        logits_ref[r0:r0 + GRP, :] = (
            jnp.dot(ln_core(x2), wout, preferred_element_type=jnp.float32)
            + out_b)


def kernel(tokens, pos_emb, tok_emb, ln1_g, ln1_b, lnf_g, lnf_b,
           wq_t, wk_t, wv_t, wo_t, cqw, cqb, ckw, ckb, cvw, cvb, wout_t):
    B = tokens.shape[0]

    wqkv_s = jnp.concatenate([wq_t, wk_t * SCALE, wv_t], axis=1)
    wqkv_g = ln1_g.reshape(DIM, 1) * wqkv_s
    qkv_bias = ln1_b.reshape(1, DIM) @ wqkv_s
    x_tab = pos_emb[:, None, :] + tok_emb[None, :, :]
    mu = jnp.mean(x_tab, axis=-1, keepdims=True)
    var = jnp.mean((x_tab - mu) ** 2, axis=-1, keepdims=True)
    xn = (x_tab - mu) * lax.rsqrt(var + LN_EPS)
    qkv_tab = xn @ wqkv_g + qkv_bias
    table = jnp.concatenate([qkv_tab, x_tab], axis=-1).reshape(L * VOCAB, 16)

    flat_idx = (jnp.arange(L, dtype=jnp.int32)[None, :] * VOCAB + tokens)
    act = jnp.take(table, flat_idx.reshape(-1), axis=0)

    wout_g = lnf_g.reshape(DIM, 1) * wout_t
    out_b = lnf_b.reshape(1, DIM) @ wout_t
    cw = jnp.concatenate([cqw, ckw, cvw], axis=1)
    cb = jnp.concatenate([cqb, ckb, cvb], axis=1)
    par = jnp.zeros((16, 16), jnp.float32)
    par = par.at[PR_CW:PR_CW + KSIZE, 0:3 * DIM].set(cw)
    par = par.at[PR_CB, 0:3 * DIM].set(cb[0])
    par = par.at[PR_OUTB, 0:VOCAB].set(out_b[0])
    par = par.at[PR_WO:PR_WO + DIM, 0:DIM].set(wo_t)
    par = par.at[PR_WOUT:PR_WOUT + DIM, 0:VOCAB].set(wout_g)

    nstep = B // G
    flops = nstep * 2 * TOKB * DIM * (2 * ATT_W + DIM + VOCAB)
    transcendentals = nstep * TOKB * ATT_W
    bytes_accessed = (B * L * (16 + VOCAB) + B * NUM_HEADS * L * L + 256) * 4

    logits_flat, attn = pl.pallas_call(
        _block_kernel,
        grid=(nstep,),
        in_specs=[
            pl.BlockSpec((TOKB, 16), lambda i: (i, 0)),
            pl.BlockSpec((16, 16), lambda i: (0, 0)),
        ],
        out_specs=(
            pl.BlockSpec((TOKB, VOCAB), lambda i: (i, 0)),
            pl.BlockSpec((G, NUM_HEADS, L, L), lambda i: (i, 0, 0, 0)),
        ),
        out_shape=(
            jax.ShapeDtypeStruct((B * L, VOCAB), jnp.float32),
            jax.ShapeDtypeStruct((B, NUM_HEADS, L, L), jnp.float32),
        ),
        compiler_params=pltpu.CompilerParams(
            dimension_semantics=("parallel",)),
        cost_estimate=pl.CostEstimate(flops=flops,
                                      transcendentals=transcendentals,
                                      bytes_accessed=bytes_accessed),
    )(act, par)

    return logits_flat.reshape(B, L, VOCAB), [attn]


# speedup vs baseline: 2.9139x; 1.5745x over previous
import jax
import jax.numpy as jnp
from jax import lax
from jax.experimental import pallas as pl
from jax.experimental.pallas import tpu as pltpu

L = 32
DIM = 4
NUM_HEADS = 2
HEAD_DIM = DIM // NUM_HEADS
VOCAB = 16
KSIZE = 3
SCALE = HEAD_DIM ** (-0.5)
LN_EPS = 1e-5

G = 16
GROUPS = G // 2
TOKB = G * L
GRP = 2 * L
ATT_W = NUM_HEADS * GRP
NEG = -1e30

PR_CW = 0
PR_CB = 3
PR_OUTB = 4
PR_WO = 5
PR_WOUT = 9


def _block_kernel(act_ref, par_ref, am_ref, km_ref, logits_ref, attn_ref):
    qkv_all = act_ref[:, 0:3 * DIM]
    x_all = act_ref[:, 3 * DIM:4 * DIM]

    cw = par_ref[PR_CW:PR_CW + KSIZE, 0:3 * DIM]
    cb = par_ref[PR_CB:PR_CB + 1, 0:3 * DIM]
    out_b = par_ref[PR_OUTB:PR_OUTB + 1, 0:VOCAB]
    wo = par_ref[PR_WO:PR_WO + DIM, 0:DIM]
    wout = par_ref[PR_WOUT:PR_WOUT + DIM, 0:VOCAB]
    am = am_ref[...]
    km = km_ref[:, 0:2 * DIM]

    pos = lax.broadcasted_iota(jnp.int32, (TOKB, 1), 0) % L
    zm1 = jnp.where(pos == 0, 0.0, pltpu.roll(qkv_all, shift=1, axis=0))
    zp1 = jnp.where(pos == L - 1, 0.0,
                    pltpu.roll(qkv_all, shift=TOKB - 1, axis=0))
    qkv_all = zm1 * cw[0:1, :] + qkv_all * cw[1:2, :] + zp1 * cw[2:3, :] + cb

    q_all = qkv_all[:, 0:DIM]
    kv_all = qkv_all[:, DIM:3 * DIM]

    s_parts = []
    v_sels = []
    for g in range(GROUPS):
        r0 = g * GRP
        kv = kv_all[r0:r0 + GRP, :]
        kv_sel = jnp.concatenate([kv, kv], axis=0) * km
        v_sels.append(kv_sel[:, DIM:2 * DIM])
        s_parts.append(
            lax.dot_general(q_all[r0:r0 + GRP, :], kv_sel[:, 0:DIM],
                            (((1,), (1,)), ((), ())),
                            preferred_element_type=jnp.float32) + am)
    s_all = jnp.concatenate(s_parts, axis=0)

    p_halves = []
    for h in range(NUM_HEADS):
        sg = s_all[:, h * GRP:(h + 1) * GRP]
        e = jnp.exp(sg - jnp.max(sg, axis=-1, keepdims=True))
        inv = pl.reciprocal(jnp.sum(e, axis=-1, keepdims=True), approx=False)
        p_halves.append(e * inv)
    p_all = jnp.concatenate(p_halves, axis=1)

    par_even = (lax.broadcasted_iota(jnp.int32, (TOKB, 1), 0) // L) % 2 == 0
    for h in range(NUM_HEADS):
        u = jnp.where(par_even,
                      p_all[:, h * GRP:h * GRP + L],
                      p_all[:, h * GRP + L:(h + 1) * GRP])
        attn_ref[:, h, :, :] = u.reshape(G, L, L)

    ctxt = jnp.concatenate(
        [jnp.dot(p_all[g * GRP:(g + 1) * GRP, :], v_sels[g],
                 preferred_element_type=jnp.float32) for g in range(GROUPS)],
        axis=0)

    x2 = x_all + jnp.dot(ctxt, wo, preferred_element_type=jnp.float32)
    mu = jnp.mean(x2, axis=-1, keepdims=True)
    var = jnp.mean((x2 - mu) ** 2, axis=-1, keepdims=True)
    xn = (x2 - mu) * lax.rsqrt(var + LN_EPS)
    logits_ref[...] = (jnp.dot(xn, wout, preferred_element_type=jnp.float32)
                       + out_b)


def kernel(tokens, pos_emb, tok_emb, ln1_g, ln1_b, lnf_g, lnf_b,
           wq_t, wk_t, wv_t, wo_t, cqw, cqb, ckw, ckb, cvw, cvb, wout_t):
    B = tokens.shape[0]

    wqkv_s = jnp.concatenate([wq_t, wk_t * SCALE, wv_t], axis=1)
    wqkv_g = ln1_g.reshape(DIM, 1) * wqkv_s
    qkv_bias = ln1_b.reshape(1, DIM) @ wqkv_s
    x_tab = pos_emb[:, None, :] + tok_emb[None, :, :]
    mu = jnp.mean(x_tab, axis=-1, keepdims=True)
    var = jnp.mean((x_tab - mu) ** 2, axis=-1, keepdims=True)
    xn = (x_tab - mu) * lax.rsqrt(var + LN_EPS)
    qkv_tab = xn @ wqkv_g + qkv_bias
    table = jnp.concatenate([qkv_tab, x_tab], axis=-1).reshape(L * VOCAB, 16)

    flat_idx = (jnp.arange(L, dtype=jnp.int32)[None, :] * VOCAB + tokens)
    act = jnp.take(table, flat_idx.reshape(-1), axis=0)

    wout_g = lnf_g.reshape(DIM, 1) * wout_t
    out_b = lnf_b.reshape(1, DIM) @ wout_t
    cw = jnp.concatenate([cqw, ckw, cvw], axis=1)
    cb = jnp.concatenate([cqb, ckb, cvb], axis=1)
    par = jnp.zeros((16, 16), jnp.float32)
    par = par.at[PR_CW:PR_CW + KSIZE, 0:3 * DIM].set(cw)
    par = par.at[PR_CB, 0:3 * DIM].set(cb[0])
    par = par.at[PR_OUTB, 0:VOCAB].set(out_b[0])
    par = par.at[PR_WO:PR_WO + DIM, 0:DIM].set(wo_t)
    par = par.at[PR_WOUT:PR_WOUT + DIM, 0:VOCAB].set(wout_g)

    r = jnp.arange(GRP)[:, None]
    c = jnp.arange(ATT_W)[None, :]
    bad = ((c // L) % 2 != r // L) | (c % L > r % L)
    am = jnp.where(bad, NEG, 0.0).astype(jnp.float32)

    rr = jnp.arange(ATT_W)[:, None]
    cc = jnp.arange(16)[None, :]
    km = ((rr // GRP) == ((cc % DIM) // HEAD_DIM)).astype(jnp.float32)

    nstep = B // G
    flops = nstep * 2 * TOKB * DIM * (2 * ATT_W + DIM + VOCAB)
    transcendentals = nstep * TOKB * ATT_W
    bytes_accessed = (B * L * (16 + VOCAB) + B * NUM_HEADS * L * L + 256) * 4

    logits_flat, attn = pl.pallas_call(
        _block_kernel,
        grid=(nstep,),
        in_specs=[
            pl.BlockSpec((TOKB, 16), lambda i: (i, 0)),
            pl.BlockSpec((16, 16), lambda i: (0, 0)),
            pl.BlockSpec((GRP, ATT_W), lambda i: (0, 0)),
            pl.BlockSpec((ATT_W, 16), lambda i: (0, 0)),
        ],
        out_specs=(
            pl.BlockSpec((TOKB, VOCAB), lambda i: (i, 0)),
            pl.BlockSpec((G, NUM_HEADS, L, L), lambda i: (i, 0, 0, 0)),
        ),
        out_shape=(
            jax.ShapeDtypeStruct((B * L, VOCAB), jnp.float32),
            jax.ShapeDtypeStruct((B, NUM_HEADS, L, L), jnp.float32),
        ),
        compiler_params=pltpu.CompilerParams(
            dimension_semantics=("parallel",)),
        cost_estimate=pl.CostEstimate(flops=flops,
                                      transcendentals=transcendentals,
                                      bytes_accessed=bytes_accessed),
    )(act, par, am, km)

    return logits_flat.reshape(B, L, VOCAB), [attn]


# speedup vs baseline: 4.1006x; 1.4073x over previous
import jax
import jax.numpy as jnp
from jax import lax
from jax.experimental import pallas as pl
from jax.experimental.pallas import tpu as pltpu

L = 32
DIM = 4
NUM_HEADS = 2
HEAD_DIM = DIM // NUM_HEADS
VOCAB = 16
KSIZE = 3
SCALE = HEAD_DIM ** (-0.5)
LN_EPS = 1e-5

G = 16
GROUPS = G // 2
TOKB = G * L
GRP = 2 * L
ATT_W = NUM_HEADS * GRP
NCLS = L * VOCAB
NEG = -1e30

PR_CW = 0
PR_CB = 3
PR_OUTB = 4
PR_WO = 5
PR_WOUT = 9


def _block_kernel(tok_ref, tab_ref, par_ref, am_ref, km_ref,
                  logits_ref, attn_ref):
    idx = tok_ref[0]
    li = lax.broadcasted_iota(jnp.int32, (TOKB, 1), 0) % L
    cls = idx + li * VOCAB
    col = lax.broadcasted_iota(jnp.int32, (TOKB, NCLS), 1)
    onehot = jnp.where(cls == col, 1.0, 0.0)
    act = jnp.dot(onehot, tab_ref[...],
                  preferred_element_type=jnp.float32)

    qkv_all = act[:, 0:3 * DIM]
    x_all = act[:, 3 * DIM:4 * DIM]

    cw = par_ref[PR_CW:PR_CW + KSIZE, 0:3 * DIM]
    cb = par_ref[PR_CB:PR_CB + 1, 0:3 * DIM]
    out_b = par_ref[PR_OUTB:PR_OUTB + 1, 0:VOCAB]
    wo = par_ref[PR_WO:PR_WO + DIM, 0:DIM]
    wout = par_ref[PR_WOUT:PR_WOUT + DIM, 0:VOCAB]
    am = am_ref[...]
    km = km_ref[:, 0:2 * DIM]

    pos = lax.broadcasted_iota(jnp.int32, (TOKB, 1), 0) % L
    zm1 = jnp.where(pos == 0, 0.0, pltpu.roll(qkv_all, shift=1, axis=0))
    zp1 = jnp.where(pos == L - 1, 0.0,
                    pltpu.roll(qkv_all, shift=TOKB - 1, axis=0))
    qkv_all = zm1 * cw[0:1, :] + qkv_all * cw[1:2, :] + zp1 * cw[2:3, :] + cb

    q_all = qkv_all[:, 0:DIM]
    kv_all = qkv_all[:, DIM:3 * DIM]

    s_parts = []
    v_sels = []
    for g in range(GROUPS):
        r0 = g * GRP
        kv = kv_all[r0:r0 + GRP, :]
        kv_sel = jnp.concatenate([kv, kv], axis=0) * km
        v_sels.append(kv_sel[:, DIM:2 * DIM])
        s_parts.append(
            lax.dot_general(q_all[r0:r0 + GRP, :], kv_sel[:, 0:DIM],
                            (((1,), (1,)), ((), ())),
                            preferred_element_type=jnp.float32) + am)
    s_all = jnp.concatenate(s_parts, axis=0)

    p_halves = []
    for h in range(NUM_HEADS):
        sg = s_all[:, h * GRP:(h + 1) * GRP]
        e = jnp.exp(sg - jnp.max(sg, axis=-1, keepdims=True))
        inv = pl.reciprocal(jnp.sum(e, axis=-1, keepdims=True), approx=False)
        p_halves.append(e * inv)
    p_all = jnp.concatenate(p_halves, axis=1)

    par_even = (lax.broadcasted_iota(jnp.int32, (TOKB, 1), 0) // L) % 2 == 0
    for h in range(NUM_HEADS):
        u = jnp.where(par_even,
                      p_all[:, h * GRP:h * GRP + L],
                      p_all[:, h * GRP + L:(h + 1) * GRP])
        attn_ref[:, h, :, :] = u.reshape(G, L, L)

    ctxt = jnp.concatenate(
        [jnp.dot(p_all[g * GRP:(g + 1) * GRP, :], v_sels[g],
                 preferred_element_type=jnp.float32) for g in range(GROUPS)],
        axis=0)

    x2 = x_all + jnp.dot(ctxt, wo, preferred_element_type=jnp.float32)
    mu = jnp.mean(x2, axis=-1, keepdims=True)
    var = jnp.mean((x2 - mu) ** 2, axis=-1, keepdims=True)
    xn = (x2 - mu) * lax.rsqrt(var + LN_EPS)
    logits_ref[...] = (jnp.dot(xn, wout, preferred_element_type=jnp.float32)
                       + out_b)


def kernel(tokens, pos_emb, tok_emb, ln1_g, ln1_b, lnf_g, lnf_b,
           wq_t, wk_t, wv_t, wo_t, cqw, cqb, ckw, ckb, cvw, cvb, wout_t):
    B = tokens.shape[0]

    wqkv_s = jnp.concatenate([wq_t, wk_t * SCALE, wv_t], axis=1)
    wqkv_g = ln1_g.reshape(DIM, 1) * wqkv_s
    qkv_bias = ln1_b.reshape(1, DIM) @ wqkv_s
    x_tab = pos_emb[:, None, :] + tok_emb[None, :, :]
    mu = jnp.mean(x_tab, axis=-1, keepdims=True)
    var = jnp.mean((x_tab - mu) ** 2, axis=-1, keepdims=True)
    xn = (x_tab - mu) * lax.rsqrt(var + LN_EPS)
    qkv_tab = xn @ wqkv_g + qkv_bias
    table = jnp.concatenate([qkv_tab, x_tab], axis=-1).reshape(NCLS, 16)

    nstep = B // G
    tok3 = tokens.reshape(nstep, TOKB, 1)

    wout_g = lnf_g.reshape(DIM, 1) * wout_t
    out_b = lnf_b.reshape(1, DIM) @ wout_t
    cw = jnp.concatenate([cqw, ckw, cvw], axis=1)
    cb = jnp.concatenate([cqb, ckb, cvb], axis=1)
    par = jnp.zeros((16, 16), jnp.float32)
    par = par.at[PR_CW:PR_CW + KSIZE, 0:3 * DIM].set(cw)
    par = par.at[PR_CB, 0:3 * DIM].set(cb[0])
    par = par.at[PR_OUTB, 0:VOCAB].set(out_b[0])
    par = par.at[PR_WO:PR_WO + DIM, 0:DIM].set(wo_t)
    par = par.at[PR_WOUT:PR_WOUT + DIM, 0:VOCAB].set(wout_g)

    r = jnp.arange(GRP)[:, None]
    c = jnp.arange(ATT_W)[None, :]
    bad = ((c // L) % 2 != r // L) | (c % L > r % L)
    am = jnp.where(bad, NEG, 0.0).astype(jnp.float32)

    rr = jnp.arange(ATT_W)[:, None]
    cc = jnp.arange(16)[None, :]
    km = ((rr // GRP) == ((cc % DIM) // HEAD_DIM)).astype(jnp.float32)

    flops = nstep * 2 * TOKB * (NCLS * 16 + DIM * (2 * ATT_W + DIM + VOCAB))
    transcendentals = nstep * TOKB * ATT_W
    bytes_accessed = (B * L * (1 + VOCAB) + B * NUM_HEADS * L * L + 8448) * 4

    logits_flat, attn = pl.pallas_call(
        _block_kernel,
        grid=(nstep,),
        in_specs=[
            pl.BlockSpec((1, TOKB, 1), lambda i: (i, 0, 0)),
            pl.BlockSpec((NCLS, 16), lambda i: (0, 0)),
            pl.BlockSpec((16, 16), lambda i: (0, 0)),
            pl.BlockSpec((GRP, ATT_W), lambda i: (0, 0)),
            pl.BlockSpec((ATT_W, 16), lambda i: (0, 0)),
        ],
        out_specs=(
            pl.BlockSpec((TOKB, VOCAB), lambda i: (i, 0)),
            pl.BlockSpec((G, NUM_HEADS, L, L), lambda i: (i, 0, 0, 0)),
        ),
        out_shape=(
            jax.ShapeDtypeStruct((B * L, VOCAB), jnp.float32),
            jax.ShapeDtypeStruct((B, NUM_HEADS, L, L), jnp.float32),
        ),
        compiler_params=pltpu.CompilerParams(
            dimension_semantics=("parallel",)),
        cost_estimate=pl.CostEstimate(flops=flops,
                                      transcendentals=transcendentals,
                                      bytes_accessed=bytes_accessed),
    )(tok3, table, par, am, km)

    return logits_flat.reshape(B, L, VOCAB), [attn]
